# baseline (speedup 1.0000x reference)
import functools

import jax
import jax.numpy as jnp
from jax import lax
from jax.experimental import pallas as pl
from jax.experimental.pallas import tpu as pltpu
from jax.experimental.pallas import tpu_sc as plsc

N = 6_400_000
S = 100_000
S_PAD = 100_352
SLICE = S_PAD // 16
P = 4_000
ST = P // 16
PER_TILE = N // 32
N_WIN = PER_TILE // P
NBUF = 2
DRAIN_BLK = 256
UNROLL = 4


def _sc_segsum(ids, vals):
    mesh = plsc.VectorSubcoreMesh(core_axis_name="c", subcore_axis_name="s")

    @functools.partial(
        pl.kernel,
        mesh=mesh,
        out_type=jax.ShapeDtypeStruct((2, S_PAD), jnp.float32),
        compiler_params=pltpu.CompilerParams(needs_layout_passes=False),
        scratch_types=(
            [pltpu.VMEM((P,), jnp.int32) for _ in range(NBUF)]
            + [pltpu.VMEM((P,), jnp.float32) for _ in range(NBUF)]
            + [pltpu.VMEM((S_PAD,), jnp.float32),
               pltpu.VMEM((DRAIN_BLK,), jnp.int32),
               pltpu.VMEM((16,), jnp.int32),
               pltpu.VMEM((16,), jnp.int32),
               pltpu.VMEM_SHARED((S_PAD,), jnp.float32)]
            + [pltpu.SemaphoreType.DMA for _ in range(NBUF)]
        ),
    )
    def k(ids_hbm, vals_hbm, out_hbm, *scratch):
        idx_bufs = scratch[:NBUF]
        val_bufs = scratch[NBUF:2 * NBUF]
        acc_t = scratch[2 * NBUF]
        drain_idx = scratch[2 * NBUF + 1]
        lo_v = scratch[2 * NBUF + 2]
        hi_v = scratch[2 * NBUF + 3]
        acc_sh = scratch[2 * NBUF + 4]
        sem_g = scratch[2 * NBUF + 5:]

        c = lax.axis_index("c")
        s = lax.axis_index("s")
        wid = c * 16 + s
        base = wid * PER_TILE
        ji = lax.iota(jnp.int32, 16)
        bidx = ji * ST
        zeros16 = jnp.zeros((16,), jnp.float32)

        def start_gather(j, b):
            off = base + j * P
            pltpu.async_copy(ids_hbm.at[pl.ds(off, P)], idx_bufs[b], sem_g[b])
            pltpu.async_copy(vals_hbm.at[pl.ds(off, P)], val_bufs[b], sem_g[b])

        def wait_gather(b):
            pltpu.make_async_copy(ids_hbm.at[pl.ds(0, P)], idx_bufs[b],
                                  sem_g[b]).wait()
            pltpu.make_async_copy(vals_hbm.at[pl.ds(0, P)], val_bufs[b],
                                  sem_g[b]).wait()

        for b in range(NBUF - 1):
            start_gather(b, b)

        @plsc.parallel_loop(0, S_PAD // 16, unroll=8)
        def _(i):
            acc_t[pl.ds(i * 16, 16)] = zeros16
        pltpu.sync_copy(acc_t.at[pl.ds(0, SLICE)],
                        acc_sh.at[pl.ds(s * SLICE, SLICE)])
        plsc.subcore_barrier()

        def process_window(b):
            idb, vlb = idx_bufs[b], val_bufs[b]
            prev0 = plsc.load_gather(idb, [bidx])
            run0 = plsc.load_gather(vlb, [bidx])

            @plsc.parallel_loop(1, ST, unroll=UNROLL, carry=(prev0, run0))
            def final(k_, carry):
                prev, run = carry
                iv = bidx + k_
                cur = plsc.load_gather(idb, [iv])
                v = plsc.load_gather(vlb, [iv])
                ch = cur != prev
                plsc.addupdate_scatter(acc_t, [prev], run, mask=ch)
                run = jnp.where(ch, v, run + v)
                return cur, run

            prev, run = final
            plsc.addupdate_scatter(acc_t, [prev], run)

        n_outer = N_WIN // NBUF

        def win_body(g, _):
            for b in range(NBUF):
                j = NBUF * g + b
                wait_gather(b)
                process_window(b)

                @pl.when(j + NBUF - 1 < N_WIN)
                def _():
                    start_gather(j + NBUF - 1, (b + NBUF - 1) % NBUF)
            return _
        lax.fori_loop(0, n_outer, win_body, None)

        pltpu.sync_copy(ids_hbm.at[pl.ds(base, 16)], lo_v)
        pltpu.sync_copy(ids_hbm.at[pl.ds(base + PER_TILE - 16, 16)], hi_v)
        lo = jnp.min(lo_v[...])
        hi = jnp.max(hi_v[...])
        lo_blk = (lo // DRAIN_BLK) * DRAIN_BLK
        n_blk = (hi - lo_blk) // DRAIN_BLK + 1

        def drain_body(t, _):
            bs = lo_blk + t * DRAIN_BLK
            for i in range(DRAIN_BLK // 16):
                drain_idx[pl.ds(i * 16, 16)] = ji + (bs + i * 16)
            pltpu.sync_copy(acc_t.at[pl.ds(bs, DRAIN_BLK)],
                            acc_sh.at[drain_idx], add=True)
            return _
        lax.fori_loop(0, n_blk, drain_body, None)
        plsc.subcore_barrier()

        pltpu.sync_copy(acc_sh.at[pl.ds(s * SLICE, SLICE)],
                        out_hbm.at[c, pl.ds(s * SLICE, SLICE)])

    return k(ids, vals)


def _tc_combine(partials):
    def body(p_ref, o_ref):
        o_ref[...] = p_ref[0] + p_ref[1]

    return pl.pallas_call(
        body,
        out_shape=jax.ShapeDtypeStruct((S_PAD,), jnp.float32),
    )(partials)


@jax.jit
def kernel(ind_1, output):
    ids = jnp.reshape(ind_1, (N,))
    vals = jnp.reshape(output, (N,))
    partials = _sc_segsum(ids, vals)
    return _tc_combine(partials)[:S]


# speedup vs baseline: 1.7244x; 1.7244x over previous
import functools

import jax
import jax.numpy as jnp
from jax import lax
from jax.experimental import pallas as pl
from jax.experimental.pallas import tpu as pltpu
from jax.experimental.pallas import tpu_sc as plsc

N = 6_400_000
S = 100_000
S_PAD = 100_352
SLICE = S_PAD // 16
P = 2_000
ST = P // 16
PER_TILE = N // 32
N_WIN = PER_TILE // P
NBUF = 5
DRAIN_BLK = 256
UNROLL = 4


def _sc_segsum(ids, vals):
    mesh = plsc.VectorSubcoreMesh(core_axis_name="c", subcore_axis_name="s")

    @functools.partial(
        pl.kernel,
        mesh=mesh,
        out_type=jax.ShapeDtypeStruct((2, S_PAD), jnp.float32),
        compiler_params=pltpu.CompilerParams(needs_layout_passes=False),
        scratch_types=(
            [pltpu.VMEM((P,), jnp.int32) for _ in range(NBUF)]
            + [pltpu.VMEM((P,), jnp.float32) for _ in range(NBUF)]
            + [pltpu.VMEM((S_PAD,), jnp.float32),
               pltpu.VMEM((DRAIN_BLK,), jnp.int32),
               pltpu.VMEM((16,), jnp.int32),
               pltpu.VMEM((16,), jnp.int32),
               pltpu.VMEM_SHARED((S_PAD,), jnp.float32)]
            + [pltpu.SemaphoreType.DMA for _ in range(NBUF)]
        ),
    )
    def k(ids_hbm, vals_hbm, out_hbm, *scratch):
        idx_bufs = scratch[:NBUF]
        val_bufs = scratch[NBUF:2 * NBUF]
        acc_t = scratch[2 * NBUF]
        drain_idx = scratch[2 * NBUF + 1]
        lo_v = scratch[2 * NBUF + 2]
        hi_v = scratch[2 * NBUF + 3]
        acc_sh = scratch[2 * NBUF + 4]
        sem_g = scratch[2 * NBUF + 5:]

        c = lax.axis_index("c")
        s = lax.axis_index("s")
        wid = c * 16 + s
        base = wid * PER_TILE
        ji = lax.iota(jnp.int32, 16)
        bidx = ji * ST
        zeros16 = jnp.zeros((16,), jnp.float32)

        def start_gather(j, b):
            off = base + j * P
            pltpu.async_copy(ids_hbm.at[pl.ds(off, P)], idx_bufs[b], sem_g[b])
            pltpu.async_copy(vals_hbm.at[pl.ds(off, P)], val_bufs[b], sem_g[b])

        def wait_gather(b):
            pltpu.make_async_copy(ids_hbm.at[pl.ds(0, P)], idx_bufs[b],
                                  sem_g[b]).wait()
            pltpu.make_async_copy(vals_hbm.at[pl.ds(0, P)], val_bufs[b],
                                  sem_g[b]).wait()

        for b in range(NBUF - 1):
            start_gather(b, b)

        @plsc.parallel_loop(0, S_PAD // 16, unroll=8)
        def _(i):
            acc_t[pl.ds(i * 16, 16)] = zeros16
        pltpu.sync_copy(acc_t.at[pl.ds(0, SLICE)],
                        acc_sh.at[pl.ds(s * SLICE, SLICE)])
        plsc.subcore_barrier()

        def process_window(b):
            idb, vlb = idx_bufs[b], val_bufs[b]
            prev0 = plsc.load_gather(idb, [bidx])
            run0 = plsc.load_gather(vlb, [bidx])

            @plsc.parallel_loop(1, ST, unroll=UNROLL, carry=(prev0, run0))
            def final(k_, carry):
                prev, run = carry
                iv = bidx + k_
                cur = plsc.load_gather(idb, [iv])
                v = plsc.load_gather(vlb, [iv])
                ch = cur != prev
                plsc.addupdate_scatter(acc_t, [prev], run, mask=ch)
                run = jnp.where(ch, v, run + v)
                return cur, run

            prev, run = final
            plsc.addupdate_scatter(acc_t, [prev], run)

        n_outer = N_WIN // NBUF

        def win_body(g, _):
            for b in range(NBUF):
                j = NBUF * g + b
                wait_gather(b)
                process_window(b)

                @pl.when(j + NBUF - 1 < N_WIN)
                def _():
                    start_gather(j + NBUF - 1, (b + NBUF - 1) % NBUF)
            return _
        lax.fori_loop(0, n_outer, win_body, None)

        pltpu.sync_copy(ids_hbm.at[pl.ds(base, 16)], lo_v)
        pltpu.sync_copy(ids_hbm.at[pl.ds(base + PER_TILE - 16, 16)], hi_v)
        lo = jnp.min(lo_v[...])
        hi = jnp.max(hi_v[...])
        lo_blk = (lo // DRAIN_BLK) * DRAIN_BLK
        n_blk = (hi - lo_blk) // DRAIN_BLK + 1

        def drain_body(t, _):
            bs = lo_blk + t * DRAIN_BLK
            for i in range(DRAIN_BLK // 16):
                drain_idx[pl.ds(i * 16, 16)] = ji + (bs + i * 16)
            pltpu.sync_copy(acc_t.at[pl.ds(bs, DRAIN_BLK)],
                            acc_sh.at[drain_idx], add=True)
            return _
        lax.fori_loop(0, n_blk, drain_body, None)
        plsc.subcore_barrier()

        pltpu.sync_copy(acc_sh.at[pl.ds(s * SLICE, SLICE)],
                        out_hbm.at[c, pl.ds(s * SLICE, SLICE)])

    return k(ids, vals)


def _tc_combine(partials):
    def body(p_ref, o_ref):
        o_ref[...] = p_ref[0] + p_ref[1]

    return pl.pallas_call(
        body,
        out_shape=jax.ShapeDtypeStruct((S_PAD,), jnp.float32),
    )(partials)


@jax.jit
def kernel(ind_1, output):
    ids = jnp.reshape(ind_1, (N,))
    vals = jnp.reshape(output, (N,))
    partials = _sc_segsum(ids, vals)
    return _tc_combine(partials)[:S]
